# baseline (device time: 771351 ns/iter reference)
import jax
import jax.numpy as jnp
import numpy as np
from jax import lax
from jax.experimental import pallas as pl
from jax.experimental.pallas import tpu as pltpu

N_DEV = 16


def _ring_order() -> list[int]:
    try:
        coords = sorted(
            tuple(d.coords)
            for d in jax.devices()
            if getattr(d, "core_on_chip", 1) == 1
        )
    except Exception:
        return list(range(N_DEV))
    if len(coords) != N_DEV:
        return list(range(N_DEV))
    snake = []
    zs = sorted({c[2] for c in coords})
    for z in zs:
        plane = [c for c in coords if c[2] == z]
        ys = sorted({c[1] for c in plane})
        for yi, y in enumerate(ys):
            row = sorted((c for c in plane if c[1] == y), reverse=bool(yi % 2))
            snake.extend(row)
    logical = {c: i for i, c in enumerate(snake)}
    xs = {c[0] for c in coords}
    ys = {c[1] for c in coords}
    if not (xs == {0, 1} and ys == {0, 1} and len(zs) * 4 == N_DEV):
        return list(range(N_DEV))
    cycle = (
        [(0, 0, z) for z in zs]
        + [(0, 1, z) for z in reversed(zs)]
        + [(1, 1, z) for z in zs]
        + [(1, 0, z) for z in reversed(zs)]
    )
    return [logical[c] for c in cycle]


def kernel(x, w_mat):
    m_per, k = x.shape
    k2, n_per = w_mat.shape
    assert k == k2

    ring = np.asarray(_ring_order(), dtype=np.int32)
    inv = np.empty(N_DEV, dtype=np.int32)
    inv[ring] = np.arange(N_DEV, dtype=np.int32)

    my = lax.axis_index("i")
    r_pos = jnp.asarray(inv)[my]
    ring_j = jnp.asarray(ring)
    succ = ring_j[(r_pos + 1) % N_DEV]
    pred = ring_j[(r_pos - 1) % N_DEV]
    hops = jnp.arange(N_DEV - 1, dtype=jnp.int32)
    origins = ring_j[jnp.mod(r_pos - 1 - hops, N_DEV)]
    meta = jnp.concatenate(
        [succ[None], pred[None], origins]
    ).astype(jnp.int32)

    def body(
        meta_ref,
        x_ref,
        w_ref,
        out_ref,
        comm_ref,
        send_sems,
        recv_sems,
        amax_ref,
        amax_send_sems,
        amax_recv_sems,
    ):
        succ_id = meta_ref[0]
        pred_id = meta_ref[1]
        my_id = lax.axis_index("i")

        barrier_sem = pltpu.get_barrier_semaphore()
        pl.semaphore_signal(
            barrier_sem, inc=1, device_id=succ_id,
            device_id_type=pl.DeviceIdType.LOGICAL,
        )
        pl.semaphore_signal(
            barrier_sem, inc=1, device_id=pred_id,
            device_id_type=pl.DeviceIdType.LOGICAL,
        )
        pl.semaphore_wait(barrier_sem, 2)

        comm_ref[0] = x_ref[...]
        y0 = jnp.dot(x_ref[...], w_ref[...], preferred_element_type=jnp.float32)
        out_ref[pl.ds(my_id * m_per, m_per), :] = y0
        run_amax = jnp.max(y0)

        for h in range(N_DEV - 1):
            send_slot = h % 2
            recv_slot = (h + 1) % 2
            rdma = pltpu.make_async_remote_copy(
                src_ref=comm_ref.at[send_slot],
                dst_ref=comm_ref.at[recv_slot],
                send_sem=send_sems.at[send_slot],
                recv_sem=recv_sems.at[recv_slot],
                device_id=succ_id,
                device_id_type=pl.DeviceIdType.LOGICAL,
            )
            rdma.start()
            rdma.wait()
            chunk = comm_ref[recv_slot]
            y = jnp.dot(chunk, w_ref[...], preferred_element_type=jnp.float32)
            origin = meta_ref[2 + h]
            out_ref[pl.ds(origin * m_per, m_per), :] = y
            run_amax = jnp.maximum(run_amax, jnp.max(y))

        run_amax = jnp.maximum(run_amax, 0.0)

        for h in range(N_DEV - 1):
            send_slot = h % 2
            recv_slot = (h + 1) % 2
            amax_ref[send_slot] = jnp.full((8, 128), run_amax, jnp.float32)
            rdma = pltpu.make_async_remote_copy(
                src_ref=amax_ref.at[send_slot],
                dst_ref=amax_ref.at[recv_slot],
                send_sem=amax_send_sems.at[send_slot],
                recv_sem=amax_recv_sems.at[recv_slot],
                device_id=succ_id,
                device_id_type=pl.DeviceIdType.LOGICAL,
            )
            rdma.start()
            rdma.wait()
            run_amax = jnp.maximum(run_amax, amax_ref[recv_slot, 0, 0])

        scale = run_amax / 448.0
        y = jnp.maximum(out_ref[...], 0.0)
        q = jnp.minimum(y / scale, 448.0).astype(jnp.float8_e4m3fn)
        out_ref[...] = q.astype(jnp.float32) * scale

    return pl.pallas_call(
        body,
        out_shape=jax.ShapeDtypeStruct((N_DEV * m_per, n_per), jnp.float32),
        in_specs=[
            pl.BlockSpec(memory_space=pltpu.SMEM),
            pl.BlockSpec(memory_space=pltpu.VMEM),
            pl.BlockSpec(memory_space=pltpu.VMEM),
        ],
        out_specs=pl.BlockSpec(memory_space=pltpu.VMEM),
        scratch_shapes=[
            pltpu.VMEM((2, m_per, k), jnp.float32),
            pltpu.SemaphoreType.DMA((2,)),
            pltpu.SemaphoreType.DMA((2,)),
            pltpu.VMEM((2, 8, 128), jnp.float32),
            pltpu.SemaphoreType.DMA((2,)),
            pltpu.SemaphoreType.DMA((2,)),
        ],
        compiler_params=pltpu.CompilerParams(collective_id=0),
    )(meta, x, w_mat)


# device time: 401096 ns/iter; 1.9231x vs baseline; 1.9231x over previous
import jax
import jax.numpy as jnp
import numpy as np
from jax import lax
from jax.experimental import pallas as pl
from jax.experimental.pallas import tpu as pltpu

N_DEV = 16
N_HOPS = N_DEV // 2


def _ring_order() -> list[int]:
    try:
        coords = sorted(
            tuple(d.coords)
            for d in jax.devices()
            if getattr(d, "core_on_chip", 1) == 1
        )
    except Exception:
        return list(range(N_DEV))
    if len(coords) != N_DEV:
        return list(range(N_DEV))
    snake = []
    zs = sorted({c[2] for c in coords})
    for z in zs:
        plane = [c for c in coords if c[2] == z]
        ys = sorted({c[1] for c in plane})
        for yi, y in enumerate(ys):
            row = sorted((c for c in plane if c[1] == y), reverse=bool(yi % 2))
            snake.extend(row)
    logical = {c: i for i, c in enumerate(snake)}
    xs = {c[0] for c in coords}
    ys = {c[1] for c in coords}
    if not (xs == {0, 1} and ys == {0, 1} and len(zs) * 4 == N_DEV):
        return list(range(N_DEV))
    cycle = (
        [(0, 0, z) for z in zs]
        + [(0, 1, z) for z in reversed(zs)]
        + [(1, 1, z) for z in zs]
        + [(1, 0, z) for z in reversed(zs)]
    )
    return [logical[c] for c in cycle]


def kernel(x, w_mat):
    m_per, k = x.shape
    k2, n_per = w_mat.shape
    assert k == k2
    half = m_per // 2

    ring = np.asarray(_ring_order(), dtype=np.int32)
    inv = np.empty(N_DEV, dtype=np.int32)
    inv[ring] = np.arange(N_DEV, dtype=np.int32)

    my = lax.axis_index("i")
    r_pos = jnp.asarray(inv)[my]
    ring_j = jnp.asarray(ring)
    succ = ring_j[(r_pos + 1) % N_DEV]
    pred = ring_j[(r_pos - 1) % N_DEV]
    hops = jnp.arange(N_HOPS, dtype=jnp.int32)
    origins_f = ring_j[jnp.mod(r_pos - 1 - hops, N_DEV)]
    origins_b = ring_j[jnp.mod(r_pos + 1 + hops, N_DEV)]
    meta = jnp.concatenate(
        [succ[None], pred[None], origins_f, origins_b]
    ).astype(jnp.int32)

    def body(
        meta_ref,
        x_ref,
        w_ref,
        out_ref,
        comm_f,
        f_send_sems,
        f_recv_sems,
        comm_b,
        b_send_sems,
        b_recv_sems,
        amax_ref,
        amax_send_sems,
        amax_recv_sems,
    ):
        succ_id = meta_ref[0]
        pred_id = meta_ref[1]
        my_id = lax.axis_index("i")

        barrier_sem = pltpu.get_barrier_semaphore()
        pl.semaphore_signal(
            barrier_sem, inc=1, device_id=succ_id,
            device_id_type=pl.DeviceIdType.LOGICAL,
        )
        pl.semaphore_signal(
            barrier_sem, inc=1, device_id=pred_id,
            device_id_type=pl.DeviceIdType.LOGICAL,
        )
        pl.semaphore_wait(barrier_sem, 2)

        comm_f[0] = x_ref[...]
        comm_b[0] = x_ref[...]

        def gemm_store(chunk, row_start, amax):
            y = jnp.dot(chunk, w_ref[...], preferred_element_type=jnp.float32)
            out_ref[pl.ds(row_start, chunk.shape[0]), :] = y
            return jnp.maximum(amax, jnp.max(y))

        run_amax = jnp.float32(0.0)
        for h in range(N_HOPS):
            s = h % 2
            rs = (h + 1) % 2
            if h < N_HOPS - 1:
                rdma_f = pltpu.make_async_remote_copy(
                    src_ref=comm_f.at[s], dst_ref=comm_f.at[rs],
                    send_sem=f_send_sems.at[s], recv_sem=f_recv_sems.at[rs],
                    device_id=succ_id, device_id_type=pl.DeviceIdType.LOGICAL,
                )
                rdma_b = pltpu.make_async_remote_copy(
                    src_ref=comm_b.at[s], dst_ref=comm_b.at[rs],
                    send_sem=b_send_sems.at[s], recv_sem=b_recv_sems.at[rs],
                    device_id=pred_id, device_id_type=pl.DeviceIdType.LOGICAL,
                )
            else:
                rdma_f = pltpu.make_async_remote_copy(
                    src_ref=comm_f.at[s].at[pl.ds(0, half)],
                    dst_ref=comm_f.at[rs].at[pl.ds(0, half)],
                    send_sem=f_send_sems.at[s], recv_sem=f_recv_sems.at[rs],
                    device_id=succ_id, device_id_type=pl.DeviceIdType.LOGICAL,
                )
                rdma_b = pltpu.make_async_remote_copy(
                    src_ref=comm_b.at[s].at[pl.ds(half, half)],
                    dst_ref=comm_b.at[rs].at[pl.ds(half, half)],
                    send_sem=b_send_sems.at[s], recv_sem=b_recv_sems.at[rs],
                    device_id=pred_id, device_id_type=pl.DeviceIdType.LOGICAL,
                )
            rdma_f.start()
            rdma_b.start()
            if h == 0:
                y0 = jnp.dot(
                    x_ref[...], w_ref[...], preferred_element_type=jnp.float32
                )
                out_ref[pl.ds(my_id * m_per, m_per), :] = y0
                run_amax = jnp.max(y0)
            else:
                of = meta_ref[2 + h - 1]
                run_amax = gemm_store(comm_f[s], of * m_per, run_amax)
                ob = meta_ref[2 + N_HOPS + h - 1]
                run_amax = gemm_store(comm_b[s], ob * m_per, run_amax)
            rdma_f.wait()
            rdma_b.wait()

        of = meta_ref[2 + N_HOPS - 1]
        run_amax = gemm_store(comm_f[0, : half], of * m_per, run_amax)
        ob = meta_ref[2 + 2 * N_HOPS - 1]
        run_amax = gemm_store(comm_b[0, half:], ob * m_per + half, run_amax)

        run_amax = jnp.maximum(run_amax, 0.0)

        for h in range(N_DEV - 1):
            s = h % 2
            rs = (h + 1) % 2
            amax_ref[s] = jnp.full((8, 128), run_amax, jnp.float32)
            rdma = pltpu.make_async_remote_copy(
                src_ref=amax_ref.at[s], dst_ref=amax_ref.at[rs],
                send_sem=amax_send_sems.at[s], recv_sem=amax_recv_sems.at[rs],
                device_id=succ_id, device_id_type=pl.DeviceIdType.LOGICAL,
            )
            rdma.start()
            rdma.wait()
            run_amax = jnp.maximum(run_amax, amax_ref[rs, 0, 0])

        scale = run_amax / 448.0
        y = jnp.maximum(out_ref[...], 0.0)
        q = jnp.minimum(y / scale, 448.0).astype(jnp.float8_e4m3fn)
        out_ref[...] = q.astype(jnp.float32) * scale

    return pl.pallas_call(
        body,
        out_shape=jax.ShapeDtypeStruct((N_DEV * m_per, n_per), jnp.float32),
        in_specs=[
            pl.BlockSpec(memory_space=pltpu.SMEM),
            pl.BlockSpec(memory_space=pltpu.VMEM),
            pl.BlockSpec(memory_space=pltpu.VMEM),
        ],
        out_specs=pl.BlockSpec(memory_space=pltpu.VMEM),
        scratch_shapes=[
            pltpu.VMEM((2, m_per, k), jnp.float32),
            pltpu.SemaphoreType.DMA((2,)),
            pltpu.SemaphoreType.DMA((2,)),
            pltpu.VMEM((2, m_per, k), jnp.float32),
            pltpu.SemaphoreType.DMA((2,)),
            pltpu.SemaphoreType.DMA((2,)),
            pltpu.VMEM((2, 8, 128), jnp.float32),
            pltpu.SemaphoreType.DMA((2,)),
            pltpu.SemaphoreType.DMA((2,)),
        ],
        compiler_params=pltpu.CompilerParams(collective_id=0),
    )(meta, x, w_mat)


# device time: 378882 ns/iter; 2.0359x vs baseline; 1.0586x over previous
import jax
import jax.numpy as jnp
import numpy as np
from jax import lax
from jax.experimental import pallas as pl
from jax.experimental.pallas import tpu as pltpu

N_DEV = 16
N_HOPS = N_DEV // 2


def _ring_order() -> list[int]:
    try:
        coords = sorted(
            tuple(d.coords)
            for d in jax.devices()
            if getattr(d, "core_on_chip", 1) == 1
        )
    except Exception:
        return list(range(N_DEV))
    if len(coords) != N_DEV:
        return list(range(N_DEV))
    snake = []
    zs = sorted({c[2] for c in coords})
    for z in zs:
        plane = [c for c in coords if c[2] == z]
        ys = sorted({c[1] for c in plane})
        for yi, y in enumerate(ys):
            row = sorted((c for c in plane if c[1] == y), reverse=bool(yi % 2))
            snake.extend(row)
    logical = {c: i for i, c in enumerate(snake)}
    xs = {c[0] for c in coords}
    ys = {c[1] for c in coords}
    if not (xs == {0, 1} and ys == {0, 1} and len(zs) * 4 == N_DEV):
        return list(range(N_DEV))
    cycle = (
        [(0, 0, z) for z in zs]
        + [(0, 1, z) for z in reversed(zs)]
        + [(1, 1, z) for z in zs]
        + [(1, 0, z) for z in reversed(zs)]
    )
    return [logical[c] for c in cycle]


def kernel(x, w_mat):
    m_per, k = x.shape
    k2, n_per = w_mat.shape
    assert k == k2
    half = m_per // 2

    ring = np.asarray(_ring_order(), dtype=np.int32)
    inv = np.empty(N_DEV, dtype=np.int32)
    inv[ring] = np.arange(N_DEV, dtype=np.int32)

    my = lax.axis_index("i")
    r_pos = jnp.asarray(inv)[my]
    ring_j = jnp.asarray(ring)
    succ = ring_j[(r_pos + 1) % N_DEV]
    pred = ring_j[(r_pos - 1) % N_DEV]
    hops = jnp.arange(N_HOPS, dtype=jnp.int32)
    origins_f = ring_j[jnp.mod(r_pos - 1 - hops, N_DEV)]
    origins_b = ring_j[jnp.mod(r_pos + 1 + hops, N_DEV)]
    meta = jnp.concatenate(
        [succ[None], pred[None], origins_f, origins_b]
    ).astype(jnp.int32)

    def body(
        meta_ref,
        x_ref,
        w_ref,
        out_ref,
        comm_f,
        f_send_sems,
        f_recv_sems,
        comm_b,
        b_send_sems,
        b_recv_sems,
        amax_ref,
        amax_send_sems,
        amax_recv_sems,
    ):
        succ_id = meta_ref[0]
        pred_id = meta_ref[1]
        my_id = lax.axis_index("i")

        barrier_sem = pltpu.get_barrier_semaphore()
        pl.semaphore_signal(
            barrier_sem, inc=1, device_id=succ_id,
            device_id_type=pl.DeviceIdType.LOGICAL,
        )
        pl.semaphore_signal(
            barrier_sem, inc=1, device_id=pred_id,
            device_id_type=pl.DeviceIdType.LOGICAL,
        )
        pl.semaphore_wait(barrier_sem, 2)

        comm_f[0] = x_ref[...]
        comm_b[0] = x_ref[...]

        def gemm_store(chunk, row_start, amax):
            y = jnp.dot(chunk, w_ref[...], preferred_element_type=jnp.float32)
            out_ref[pl.ds(row_start, chunk.shape[0]), :] = y
            return jnp.maximum(amax, jnp.max(y))

        run_amax = jnp.float32(0.0)
        for h in range(N_HOPS):
            s = h % 2
            rs = (h + 1) % 2
            if h < N_HOPS - 1:
                rdma_f = pltpu.make_async_remote_copy(
                    src_ref=comm_f.at[s], dst_ref=comm_f.at[rs],
                    send_sem=f_send_sems.at[s], recv_sem=f_recv_sems.at[rs],
                    device_id=succ_id, device_id_type=pl.DeviceIdType.LOGICAL,
                )
                rdma_b = pltpu.make_async_remote_copy(
                    src_ref=comm_b.at[s], dst_ref=comm_b.at[rs],
                    send_sem=b_send_sems.at[s], recv_sem=b_recv_sems.at[rs],
                    device_id=pred_id, device_id_type=pl.DeviceIdType.LOGICAL,
                )
            else:
                rdma_f = pltpu.make_async_remote_copy(
                    src_ref=comm_f.at[s].at[pl.ds(0, half)],
                    dst_ref=comm_f.at[rs].at[pl.ds(0, half)],
                    send_sem=f_send_sems.at[s], recv_sem=f_recv_sems.at[rs],
                    device_id=succ_id, device_id_type=pl.DeviceIdType.LOGICAL,
                )
                rdma_b = pltpu.make_async_remote_copy(
                    src_ref=comm_b.at[s].at[pl.ds(half, half)],
                    dst_ref=comm_b.at[rs].at[pl.ds(half, half)],
                    send_sem=b_send_sems.at[s], recv_sem=b_recv_sems.at[rs],
                    device_id=pred_id, device_id_type=pl.DeviceIdType.LOGICAL,
                )
            rdma_f.start()
            rdma_b.start()
            if h == 0:
                y0 = jnp.dot(
                    x_ref[...], w_ref[...], preferred_element_type=jnp.float32
                )
                out_ref[pl.ds(my_id * m_per, m_per), :] = y0
                run_amax = jnp.max(y0)
            else:
                of = meta_ref[2 + h - 1]
                run_amax = gemm_store(comm_f[s], of * m_per, run_amax)
                ob = meta_ref[2 + N_HOPS + h - 1]
                run_amax = gemm_store(comm_b[s], ob * m_per, run_amax)
            rdma_f.wait()
            rdma_b.wait()

        of = meta_ref[2 + N_HOPS - 1]
        run_amax = gemm_store(comm_f[0, : half], of * m_per, run_amax)
        ob = meta_ref[2 + 2 * N_HOPS - 1]
        run_amax = gemm_store(comm_b[0, half:], ob * m_per + half, run_amax)

        run_amax = jnp.maximum(run_amax, 0.0)

        amax_ref[my_id] = jnp.full((8, 128), run_amax, jnp.float32)
        sends = []
        for j in range(1, N_DEV):
            other = jnp.mod(my_id + j, N_DEV)
            s_rdma = pltpu.make_async_remote_copy(
                src_ref=amax_ref.at[my_id],
                dst_ref=amax_ref.at[my_id],
                send_sem=amax_send_sems.at[other],
                recv_sem=amax_recv_sems.at[my_id],
                device_id=other, device_id_type=pl.DeviceIdType.LOGICAL,
            )
            s_rdma.start()
            sends.append(s_rdma)
        for j in range(1, N_DEV):
            other = jnp.mod(my_id + j, N_DEV)
            r_rdma = pltpu.make_async_remote_copy(
                src_ref=amax_ref.at[other],
                dst_ref=amax_ref.at[other],
                send_sem=amax_send_sems.at[other],
                recv_sem=amax_recv_sems.at[other],
                device_id=other, device_id_type=pl.DeviceIdType.LOGICAL,
            )
            r_rdma.wait_recv()
        for s_rdma in sends:
            s_rdma.wait_send()
        run_amax = jnp.max(amax_ref[...])

        scale = run_amax / 448.0
        y = jnp.maximum(out_ref[...], 0.0)
        q = jnp.minimum(y / scale, 448.0).astype(jnp.float8_e4m3fn)
        out_ref[...] = q.astype(jnp.float32) * scale

    return pl.pallas_call(
        body,
        out_shape=jax.ShapeDtypeStruct((N_DEV * m_per, n_per), jnp.float32),
        in_specs=[
            pl.BlockSpec(memory_space=pltpu.SMEM),
            pl.BlockSpec(memory_space=pltpu.VMEM),
            pl.BlockSpec(memory_space=pltpu.VMEM),
        ],
        out_specs=pl.BlockSpec(memory_space=pltpu.VMEM),
        scratch_shapes=[
            pltpu.VMEM((2, m_per, k), jnp.float32),
            pltpu.SemaphoreType.DMA((2,)),
            pltpu.SemaphoreType.DMA((2,)),
            pltpu.VMEM((2, m_per, k), jnp.float32),
            pltpu.SemaphoreType.DMA((2,)),
            pltpu.SemaphoreType.DMA((2,)),
            pltpu.VMEM((N_DEV, 8, 128), jnp.float32),
            pltpu.SemaphoreType.DMA((N_DEV,)),
            pltpu.SemaphoreType.DMA((N_DEV,)),
        ],
        compiler_params=pltpu.CompilerParams(collective_id=0),
    )(meta, x, w_mat)


# device time: 367285 ns/iter; 2.1001x vs baseline; 1.0316x over previous
import jax
import jax.numpy as jnp
import numpy as np
from jax import lax
from jax.experimental import pallas as pl
from jax.experimental.pallas import tpu as pltpu

N_DEV = 16
N_MSG = N_DEV - 1
N_SLOT = 6


def _ring_order() -> list[int]:
    try:
        coords = sorted(
            tuple(d.coords)
            for d in jax.devices()
            if getattr(d, "core_on_chip", 1) == 1
        )
    except Exception:
        return list(range(N_DEV))
    if len(coords) != N_DEV:
        return list(range(N_DEV))
    snake = []
    zs = sorted({c[2] for c in coords})
    for z in zs:
        plane = [c for c in coords if c[2] == z]
        ys = sorted({c[1] for c in plane})
        for yi, y in enumerate(ys):
            row = sorted((c for c in plane if c[1] == y), reverse=bool(yi % 2))
            snake.extend(row)
    logical = {c: i for i, c in enumerate(snake)}
    xs = {c[0] for c in coords}
    ys = {c[1] for c in coords}
    if not (xs == {0, 1} and ys == {0, 1} and len(zs) * 4 == N_DEV):
        return list(range(N_DEV))
    cycle = (
        [(0, 0, z) for z in zs]
        + [(0, 1, z) for z in reversed(zs)]
        + [(1, 1, z) for z in zs]
        + [(1, 0, z) for z in reversed(zs)]
    )
    return [logical[c] for c in cycle]


def kernel(x, w_mat):
    m_per, k = x.shape
    k2, n_per = w_mat.shape
    assert k == k2
    half = m_per // 2

    ring = np.asarray(_ring_order(), dtype=np.int32)
    inv = np.empty(N_DEV, dtype=np.int32)
    inv[ring] = np.arange(N_DEV, dtype=np.int32)

    my = lax.axis_index("i")
    r_pos = jnp.asarray(inv)[my]
    ring_j = jnp.asarray(ring)
    succ = ring_j[(r_pos + 1) % N_DEV]
    pred = ring_j[(r_pos - 1) % N_DEV]
    msg = jnp.arange(N_MSG, dtype=jnp.int32)
    origins_f = ring_j[jnp.mod(r_pos - 1 - msg // 2, N_DEV)]
    origins_b = ring_j[jnp.mod(r_pos + 1 + msg // 2, N_DEV)]
    meta = jnp.concatenate(
        [succ[None], pred[None], origins_f, origins_b]
    ).astype(jnp.int32)

    off_f = [(i % 2) * half for i in range(N_MSG)]
    off_b = [(1 - i % 2) * half for i in range(N_MSG)]

    def body(
        meta_ref,
        x_ref,
        w_ref,
        out_ref,
        comm_f,
        f_send_sems,
        f_recv_sems,
        comm_b,
        b_send_sems,
        b_recv_sems,
        credit_sems,
        amax_ref,
        amax_send_sems,
        amax_recv_sems,
    ):
        succ_id = meta_ref[0]
        pred_id = meta_ref[1]
        my_id = lax.axis_index("i")

        barrier_sem = pltpu.get_barrier_semaphore()
        pl.semaphore_signal(
            barrier_sem, inc=1, device_id=succ_id,
            device_id_type=pl.DeviceIdType.LOGICAL,
        )
        pl.semaphore_signal(
            barrier_sem, inc=1, device_id=pred_id,
            device_id_type=pl.DeviceIdType.LOGICAL,
        )
        pl.semaphore_wait(barrier_sem, 2)

        dirs = {
            "f": dict(comm=comm_f, ssem=f_send_sems, rsem=f_recv_sems,
                      out_id=succ_id, in_id=pred_id, credit=0,
                      off=off_f, meta0=2),
            "b": dict(comm=comm_b, ssem=b_send_sems, rsem=b_recv_sems,
                      out_id=pred_id, in_id=succ_id, credit=1,
                      off=off_b, meta0=2 + N_MSG),
        }
        send_desc = {"f": {}, "b": {}}

        def start_send(d, j):
            dd = dirs[d]
            if j < 2:
                src = x_ref.at[pl.ds(dd["off"][0] if j == 0 else dd["off"][1],
                                     half)]
            else:
                src = dd["comm"].at[(j - 2) % N_SLOT]
            rdma = pltpu.make_async_remote_copy(
                src_ref=src,
                dst_ref=dd["comm"].at[j % N_SLOT],
                send_sem=dd["ssem"].at[j % N_SLOT],
                recv_sem=dd["rsem"].at[j % N_SLOT],
                device_id=dd["out_id"],
                device_id_type=pl.DeviceIdType.LOGICAL,
            )
            rdma.start()
            send_desc[d][j] = rdma

        def recv_desc(d, i):
            dd = dirs[d]
            return pltpu.make_async_remote_copy(
                src_ref=dd["comm"].at[i % N_SLOT],
                dst_ref=dd["comm"].at[i % N_SLOT],
                send_sem=dd["ssem"].at[i % N_SLOT],
                recv_sem=dd["rsem"].at[i % N_SLOT],
                device_id=dd["in_id"],
                device_id_type=pl.DeviceIdType.LOGICAL,
            )

        def step_comm(d, i):
            dd = dirs[d]
            recv_desc(d, i).wait_recv()
            send_desc[d][i].wait_send()
            if 2 <= i <= 2 + (N_MSG - N_SLOT - 1):
                pl.semaphore_signal(
                    credit_sems.at[dd["credit"]], inc=1,
                    device_id=dd["in_id"],
                    device_id_type=pl.DeviceIdType.LOGICAL,
                )
            if i + 2 < N_MSG:
                if i + 2 >= N_SLOT:
                    pl.semaphore_wait(credit_sems.at[dd["credit"]], 1)
                start_send(d, i + 2)

        def gemm_store(chunk, row_start, amax):
            y = jnp.dot(chunk, w_ref[...], preferred_element_type=jnp.float32)
            out_ref[pl.ds(row_start, chunk.shape[0]), :] = y
            return jnp.maximum(amax, jnp.max(y))

        for d in ("f", "b"):
            start_send(d, 0)
            start_send(d, 1)
        y0 = jnp.dot(x_ref[...], w_ref[...], preferred_element_type=jnp.float32)
        out_ref[pl.ds(my_id * m_per, m_per), :] = y0
        run_amax = jnp.max(y0)

        for i in range(N_MSG):
            step_comm("f", i)
            step_comm("b", i)
            of = meta_ref[2 + i]
            run_amax = gemm_store(
                comm_f[i % N_SLOT], of * m_per + off_f[i], run_amax
            )
            ob = meta_ref[2 + N_MSG + i]
            run_amax = gemm_store(
                comm_b[i % N_SLOT], ob * m_per + off_b[i], run_amax
            )

        run_amax = jnp.maximum(run_amax, 0.0)

        amax_ref[my_id] = jnp.full((8, 128), run_amax, jnp.float32)
        sends = []
        for j in range(1, N_DEV):
            other = jnp.mod(my_id + j, N_DEV)
            s_rdma = pltpu.make_async_remote_copy(
                src_ref=amax_ref.at[my_id],
                dst_ref=amax_ref.at[my_id],
                send_sem=amax_send_sems.at[other],
                recv_sem=amax_recv_sems.at[my_id],
                device_id=other, device_id_type=pl.DeviceIdType.LOGICAL,
            )
            s_rdma.start()
            sends.append(s_rdma)
        for j in range(1, N_DEV):
            other = jnp.mod(my_id + j, N_DEV)
            r_rdma = pltpu.make_async_remote_copy(
                src_ref=amax_ref.at[other],
                dst_ref=amax_ref.at[other],
                send_sem=amax_send_sems.at[other],
                recv_sem=amax_recv_sems.at[other],
                device_id=other, device_id_type=pl.DeviceIdType.LOGICAL,
            )
            r_rdma.wait_recv()
        for s_rdma in sends:
            s_rdma.wait_send()
        run_amax = jnp.max(amax_ref[...])

        scale = run_amax / 448.0
        y = jnp.maximum(out_ref[...], 0.0)
        q = jnp.minimum(y / scale, 448.0).astype(jnp.float8_e4m3fn)
        out_ref[...] = q.astype(jnp.float32) * scale

    return pl.pallas_call(
        body,
        out_shape=jax.ShapeDtypeStruct((N_DEV * m_per, n_per), jnp.float32),
        in_specs=[
            pl.BlockSpec(memory_space=pltpu.SMEM),
            pl.BlockSpec(memory_space=pltpu.VMEM),
            pl.BlockSpec(memory_space=pltpu.VMEM),
        ],
        out_specs=pl.BlockSpec(memory_space=pltpu.VMEM),
        scratch_shapes=[
            pltpu.VMEM((N_SLOT, half, k), jnp.float32),
            pltpu.SemaphoreType.DMA((N_SLOT,)),
            pltpu.SemaphoreType.DMA((N_SLOT,)),
            pltpu.VMEM((N_SLOT, half, k), jnp.float32),
            pltpu.SemaphoreType.DMA((N_SLOT,)),
            pltpu.SemaphoreType.DMA((N_SLOT,)),
            pltpu.SemaphoreType.REGULAR((2,)),
            pltpu.VMEM((N_DEV, 8, 128), jnp.float32),
            pltpu.SemaphoreType.DMA((N_DEV,)),
            pltpu.SemaphoreType.DMA((N_DEV,)),
        ],
        compiler_params=pltpu.CompilerParams(collective_id=0),
    )(meta, x, w_mat)


# device time: 366265 ns/iter; 2.1060x vs baseline; 1.0028x over previous
import jax
import jax.numpy as jnp
import numpy as np
from jax import lax
from jax.experimental import pallas as pl
from jax.experimental.pallas import tpu as pltpu

N_DEV = 16
N_MSG = N_DEV - 1
N_SLOT = 6


def _ring_order() -> list[int]:
    try:
        coords = sorted(
            tuple(d.coords)
            for d in jax.devices()
            if getattr(d, "core_on_chip", 1) == 1
        )
    except Exception:
        return list(range(N_DEV))
    if len(coords) != N_DEV:
        return list(range(N_DEV))
    snake = []
    zs = sorted({c[2] for c in coords})
    for z in zs:
        plane = [c for c in coords if c[2] == z]
        ys = sorted({c[1] for c in plane})
        for yi, y in enumerate(ys):
            row = sorted((c for c in plane if c[1] == y), reverse=bool(yi % 2))
            snake.extend(row)
    logical = {c: i for i, c in enumerate(snake)}
    xs = {c[0] for c in coords}
    ys = {c[1] for c in coords}
    if not (xs == {0, 1} and ys == {0, 1} and len(zs) * 4 == N_DEV):
        return list(range(N_DEV))
    cycle = (
        [(0, 0, z) for z in zs]
        + [(0, 1, z) for z in reversed(zs)]
        + [(1, 1, z) for z in zs]
        + [(1, 0, z) for z in reversed(zs)]
    )
    return [logical[c] for c in cycle]


def kernel(x, w_mat):
    m_per, k = x.shape
    k2, n_per = w_mat.shape
    assert k == k2
    half = m_per // 2

    ring = np.asarray(_ring_order(), dtype=np.int32)
    inv = np.empty(N_DEV, dtype=np.int32)
    inv[ring] = np.arange(N_DEV, dtype=np.int32)

    my = lax.axis_index("i")
    r_pos = jnp.asarray(inv)[my]
    ring_j = jnp.asarray(ring)
    succ = ring_j[(r_pos + 1) % N_DEV]
    pred = ring_j[(r_pos - 1) % N_DEV]
    msg = jnp.arange(N_MSG, dtype=jnp.int32)
    origins_f = ring_j[jnp.mod(r_pos - 1 - msg // 2, N_DEV)]
    origins_b = ring_j[jnp.mod(r_pos + 1 + msg // 2, N_DEV)]
    meta = jnp.concatenate(
        [succ[None], pred[None], origins_f, origins_b]
    ).astype(jnp.int32)

    off_f = [(i % 2) * half for i in range(N_MSG)]
    off_b = [(1 - i % 2) * half for i in range(N_MSG)]

    def body(
        meta_ref,
        x_ref,
        w_ref,
        out_ref,
        comm_f,
        f_send_sems,
        f_recv_sems,
        comm_b,
        b_send_sems,
        b_recv_sems,
        credit_sems,
        amax_ref,
        amax_send_sems,
        amax_recv_sems,
    ):
        succ_id = meta_ref[0]
        pred_id = meta_ref[1]
        my_id = lax.axis_index("i")

        barrier_sem = pltpu.get_barrier_semaphore()
        pl.semaphore_signal(
            barrier_sem, inc=1, device_id=succ_id,
            device_id_type=pl.DeviceIdType.LOGICAL,
        )
        pl.semaphore_signal(
            barrier_sem, inc=1, device_id=pred_id,
            device_id_type=pl.DeviceIdType.LOGICAL,
        )
        pl.semaphore_wait(barrier_sem, 2)

        dirs = {
            "f": dict(comm=comm_f, ssem=f_send_sems, rsem=f_recv_sems,
                      out_id=succ_id, in_id=pred_id, credit=0,
                      off=off_f, meta0=2),
            "b": dict(comm=comm_b, ssem=b_send_sems, rsem=b_recv_sems,
                      out_id=pred_id, in_id=succ_id, credit=1,
                      off=off_b, meta0=2 + N_MSG),
        }
        send_desc = {"f": {}, "b": {}}

        def start_send(d, j):
            dd = dirs[d]
            if j < 2:
                src = x_ref.at[pl.ds(dd["off"][0] if j == 0 else dd["off"][1],
                                     half)]
            else:
                src = dd["comm"].at[(j - 2) % N_SLOT]
            rdma = pltpu.make_async_remote_copy(
                src_ref=src,
                dst_ref=dd["comm"].at[j % N_SLOT],
                send_sem=dd["ssem"].at[j % N_SLOT],
                recv_sem=dd["rsem"].at[j % N_SLOT],
                device_id=dd["out_id"],
                device_id_type=pl.DeviceIdType.LOGICAL,
            )
            rdma.start()
            send_desc[d][j] = rdma

        def recv_desc(d, i):
            dd = dirs[d]
            return pltpu.make_async_remote_copy(
                src_ref=dd["comm"].at[i % N_SLOT],
                dst_ref=dd["comm"].at[i % N_SLOT],
                send_sem=dd["ssem"].at[i % N_SLOT],
                recv_sem=dd["rsem"].at[i % N_SLOT],
                device_id=dd["in_id"],
                device_id_type=pl.DeviceIdType.LOGICAL,
            )

        def step_comm(d, i):
            dd = dirs[d]
            recv_desc(d, i).wait_recv()
            send_desc[d][i].wait_send()
            if 2 <= i <= 2 + (N_MSG - N_SLOT - 1):
                pl.semaphore_signal(
                    credit_sems.at[dd["credit"]], inc=1,
                    device_id=dd["in_id"],
                    device_id_type=pl.DeviceIdType.LOGICAL,
                )
            if i + 2 < N_MSG:
                if i + 2 >= N_SLOT:
                    pl.semaphore_wait(credit_sems.at[dd["credit"]], 1)
                start_send(d, i + 2)

        def gemm_store(chunk, row_start, amax):
            y = jnp.dot(chunk, w_ref[...], preferred_element_type=jnp.float32)
            y = jnp.maximum(y, 0.0)
            out_ref[pl.ds(row_start, chunk.shape[0]), :] = y
            return jnp.maximum(amax, jnp.max(y))

        for d in ("f", "b"):
            start_send(d, 0)
            start_send(d, 1)
        y0 = jnp.dot(x_ref[...], w_ref[...], preferred_element_type=jnp.float32)
        y0 = jnp.maximum(y0, 0.0)
        out_ref[pl.ds(my_id * m_per, m_per), :] = y0
        run_amax = jnp.max(y0)

        for i in range(N_MSG):
            step_comm("f", i)
            step_comm("b", i)
            of = meta_ref[2 + i]
            run_amax = gemm_store(
                comm_f[i % N_SLOT], of * m_per + off_f[i], run_amax
            )
            ob = meta_ref[2 + N_MSG + i]
            run_amax = gemm_store(
                comm_b[i % N_SLOT], ob * m_per + off_b[i], run_amax
            )

        run_amax = jnp.maximum(run_amax, 0.0)

        amax_ref[my_id] = jnp.full((8, 128), run_amax, jnp.float32)
        sends = []
        for j in range(1, N_DEV):
            other = jnp.mod(my_id + j, N_DEV)
            s_rdma = pltpu.make_async_remote_copy(
                src_ref=amax_ref.at[my_id],
                dst_ref=amax_ref.at[my_id],
                send_sem=amax_send_sems.at[other],
                recv_sem=amax_recv_sems.at[my_id],
                device_id=other, device_id_type=pl.DeviceIdType.LOGICAL,
            )
            s_rdma.start()
            sends.append(s_rdma)
        for j in range(1, N_DEV):
            other = jnp.mod(my_id + j, N_DEV)
            r_rdma = pltpu.make_async_remote_copy(
                src_ref=amax_ref.at[other],
                dst_ref=amax_ref.at[other],
                send_sem=amax_send_sems.at[other],
                recv_sem=amax_recv_sems.at[other],
                device_id=other, device_id_type=pl.DeviceIdType.LOGICAL,
            )
            r_rdma.wait_recv()
        for s_rdma in sends:
            s_rdma.wait_send()
        run_amax = jnp.max(amax_ref[...])

        scale = run_amax / 448.0
        inv_scale = 448.0 / run_amax
        y = out_ref[...]
        q = jnp.minimum(y * inv_scale, 448.0).astype(jnp.float8_e4m3fn)
        out_ref[...] = q.astype(jnp.float32) * scale

    return pl.pallas_call(
        body,
        out_shape=jax.ShapeDtypeStruct((N_DEV * m_per, n_per), jnp.float32),
        in_specs=[
            pl.BlockSpec(memory_space=pltpu.SMEM),
            pl.BlockSpec(memory_space=pltpu.VMEM),
            pl.BlockSpec(memory_space=pltpu.VMEM),
        ],
        out_specs=pl.BlockSpec(memory_space=pltpu.VMEM),
        scratch_shapes=[
            pltpu.VMEM((N_SLOT, half, k), jnp.float32),
            pltpu.SemaphoreType.DMA((N_SLOT,)),
            pltpu.SemaphoreType.DMA((N_SLOT,)),
            pltpu.VMEM((N_SLOT, half, k), jnp.float32),
            pltpu.SemaphoreType.DMA((N_SLOT,)),
            pltpu.SemaphoreType.DMA((N_SLOT,)),
            pltpu.SemaphoreType.REGULAR((2,)),
            pltpu.VMEM((N_DEV, 8, 128), jnp.float32),
            pltpu.SemaphoreType.DMA((N_DEV,)),
            pltpu.SemaphoreType.DMA((N_DEV,)),
        ],
        compiler_params=pltpu.CompilerParams(collective_id=0),
    )(meta, x, w_mat)


# device time: 354372 ns/iter; 2.1767x vs baseline; 1.0336x over previous
import jax
import jax.numpy as jnp
import numpy as np
from jax import lax
from jax.experimental import pallas as pl
from jax.experimental.pallas import tpu as pltpu

N_DEV = 16
N_MSG = N_DEV - 2
N_SLOT = 6

M_POS = [7, 6, 13, 12, 11, 10, 1, 0, 15, 14, 5, 4, 3, 2, 9, 8]


def _ring_order() -> list[int]:
    try:
        coords = sorted(
            tuple(d.coords)
            for d in jax.devices()
            if getattr(d, "core_on_chip", 1) == 1
        )
    except Exception:
        return list(range(N_DEV))
    if len(coords) != N_DEV:
        return list(range(N_DEV))
    snake = []
    zs = sorted({c[2] for c in coords})
    for z in zs:
        plane = [c for c in coords if c[2] == z]
        ys = sorted({c[1] for c in plane})
        for yi, y in enumerate(ys):
            row = sorted((c for c in plane if c[1] == y), reverse=bool(yi % 2))
            snake.extend(row)
    logical = {c: i for i, c in enumerate(snake)}
    xs = {c[0] for c in coords}
    ys = {c[1] for c in coords}
    if not (xs == {0, 1} and ys == {0, 1} and len(zs) * 4 == N_DEV):
        return list(range(N_DEV))
    cycle = (
        [(0, 0, z) for z in zs]
        + [(0, 1, z) for z in reversed(zs)]
        + [(1, 1, z) for z in zs]
        + [(1, 0, z) for z in reversed(zs)]
    )
    return [logical[c] for c in cycle]


def kernel(x, w_mat):
    m_per, k = x.shape
    k2, n_per = w_mat.shape
    assert k == k2
    half = m_per // 2

    ring = np.asarray(_ring_order(), dtype=np.int32)
    inv = np.empty(N_DEV, dtype=np.int32)
    inv[ring] = np.arange(N_DEV, dtype=np.int32)

    my = lax.axis_index("i")
    r_pos = jnp.asarray(inv)[my]
    ring_j = jnp.asarray(ring)
    succ = ring_j[(r_pos + 1) % N_DEV]
    pred = ring_j[(r_pos - 1) % N_DEV]
    msg = jnp.arange(N_MSG, dtype=jnp.int32)
    origins_f = ring_j[jnp.mod(r_pos - 1 - msg // 2, N_DEV)]
    origins_b = ring_j[jnp.mod(r_pos + 1 + msg // 2, N_DEV)]
    partner = ring_j[jnp.asarray(np.asarray(M_POS, dtype=np.int32))[r_pos]]
    relay_cls = jnp.mod(r_pos, 4)
    anti = ring_j[jnp.mod(r_pos + N_DEV // 2, N_DEV)]
    meta = jnp.concatenate(
        [succ[None], pred[None], origins_f, origins_b,
         partner[None], relay_cls[None], anti[None]]
    ).astype(jnp.int32)

    off_f = [(i % 2) * half for i in range(N_MSG)]
    off_b = [(1 - i % 2) * half for i in range(N_MSG)]

    def body(
        meta_ref,
        x_ref,
        w_ref,
        out_ref,
        comm_f,
        f_send_sems,
        f_recv_sems,
        comm_b,
        b_send_sems,
        b_recv_sems,
        credit_sems,
        staged,
        relay_in,
        relay_send_sems,
        relay_recv_sems,
        amax_ref,
        amax_send_sems,
        amax_recv_sems,
    ):
        succ_id = meta_ref[0]
        pred_id = meta_ref[1]
        partner_id = meta_ref[2 + 2 * N_MSG]
        cls = meta_ref[2 + 2 * N_MSG + 1]
        anti_id = meta_ref[2 + 2 * N_MSG + 2]
        my_id = lax.axis_index("i")

        barrier_sem = pltpu.get_barrier_semaphore()
        for nbr in (succ_id, pred_id, partner_id):
            pl.semaphore_signal(
                barrier_sem, inc=1, device_id=nbr,
                device_id_type=pl.DeviceIdType.LOGICAL,
            )
        pl.semaphore_wait(barrier_sem, 3)

        dirs = {
            "f": dict(comm=comm_f, ssem=f_send_sems, rsem=f_recv_sems,
                      out_id=succ_id, in_id=pred_id, credit=0,
                      off=off_f, meta0=2),
            "b": dict(comm=comm_b, ssem=b_send_sems, rsem=b_recv_sems,
                      out_id=pred_id, in_id=succ_id, credit=1,
                      off=off_b, meta0=2 + N_MSG),
        }
        send_desc = {"f": {}, "b": {}}

        def start_send(d, j):
            dd = dirs[d]
            if j < 2:
                src = x_ref.at[pl.ds(dd["off"][0] if j == 0 else dd["off"][1],
                                     half)]
            else:
                src = dd["comm"].at[(j - 2) % N_SLOT]
            rdma = pltpu.make_async_remote_copy(
                src_ref=src,
                dst_ref=dd["comm"].at[j % N_SLOT],
                send_sem=dd["ssem"].at[j % N_SLOT],
                recv_sem=dd["rsem"].at[j % N_SLOT],
                device_id=dd["out_id"],
                device_id_type=pl.DeviceIdType.LOGICAL,
            )
            rdma.start()
            send_desc[d][j] = rdma

        def recv_desc(d, i):
            dd = dirs[d]
            return pltpu.make_async_remote_copy(
                src_ref=dd["comm"].at[i % N_SLOT],
                dst_ref=dd["comm"].at[i % N_SLOT],
                send_sem=dd["ssem"].at[i % N_SLOT],
                recv_sem=dd["rsem"].at[i % N_SLOT],
                device_id=dd["in_id"],
                device_id_type=pl.DeviceIdType.LOGICAL,
            )

        def step_comm(d, i):
            dd = dirs[d]
            recv_desc(d, i).wait_recv()
            send_desc[d][i].wait_send()
            if 2 <= i <= 2 + (N_MSG - N_SLOT - 1):
                pl.semaphore_signal(
                    credit_sems.at[dd["credit"]], inc=1,
                    device_id=dd["in_id"],
                    device_id_type=pl.DeviceIdType.LOGICAL,
                )
            if i + 2 < N_MSG:
                if i + 2 >= N_SLOT:
                    pl.semaphore_wait(credit_sems.at[dd["credit"]], 1)
                start_send(d, i + 2)

        def gemm_store(chunk, row_start, amax):
            y = jnp.dot(chunk, w_ref[...], preferred_element_type=jnp.float32)
            y = jnp.maximum(y, 0.0)
            out_ref[pl.ds(row_start, chunk.shape[0]), :] = y
            return jnp.maximum(amax, jnp.max(y))

        for d in ("f", "b"):
            start_send(d, 0)
            start_send(d, 1)
        y0 = jnp.dot(x_ref[...], w_ref[...], preferred_element_type=jnp.float32)
        y0 = jnp.maximum(y0, 0.0)
        out_ref[pl.ds(my_id * m_per, m_per), :] = y0
        run_amax = jnp.max(y0)

        relay_desc = [
            pltpu.make_async_remote_copy(
                src_ref=staged.at[t],
                dst_ref=relay_in.at[t],
                send_sem=relay_send_sems.at[t],
                recv_sem=relay_recv_sems.at[t],
                device_id=partner_id,
                device_id_type=pl.DeviceIdType.LOGICAL,
            )
            for t in range(2)
        ]

        for i in range(N_MSG):
            step_comm("f", i)
            step_comm("b", i)
            if i == 1:
                @pl.when(cls == 0)
                def _():
                    staged[0] = comm_f[0 % N_SLOT]
                    staged[1] = comm_f[1 % N_SLOT]

                @pl.when(cls == 3)
                def _():
                    staged[0] = comm_b[1 % N_SLOT]
                    staged[1] = comm_b[0 % N_SLOT]
            if i == 5:
                @pl.when(cls == 1)
                def _():
                    staged[0] = comm_f[4 % N_SLOT]
                    staged[1] = comm_f[5 % N_SLOT]

                @pl.when(cls == 2)
                def _():
                    staged[0] = comm_b[5 % N_SLOT]
                    staged[1] = comm_b[4 % N_SLOT]

                for t in range(2):
                    relay_desc[t].start()
            of = meta_ref[2 + i]
            run_amax = gemm_store(
                comm_f[i % N_SLOT], of * m_per + off_f[i], run_amax
            )
            ob = meta_ref[2 + N_MSG + i]
            run_amax = gemm_store(
                comm_b[i % N_SLOT], ob * m_per + off_b[i], run_amax
            )

        for t in range(2):
            relay_desc[t].wait_recv()
            run_amax = gemm_store(
                relay_in[t], anti_id * m_per + t * half, run_amax
            )
        for t in range(2):
            relay_desc[t].wait_send()

        run_amax = jnp.maximum(run_amax, 0.0)

        amax_ref[my_id] = jnp.full((8, 128), run_amax, jnp.float32)
        sends = []
        for j in range(1, N_DEV):
            other = jnp.mod(my_id + j, N_DEV)
            s_rdma = pltpu.make_async_remote_copy(
                src_ref=amax_ref.at[my_id],
                dst_ref=amax_ref.at[my_id],
                send_sem=amax_send_sems.at[other],
                recv_sem=amax_recv_sems.at[my_id],
                device_id=other, device_id_type=pl.DeviceIdType.LOGICAL,
            )
            s_rdma.start()
            sends.append(s_rdma)
        for j in range(1, N_DEV):
            other = jnp.mod(my_id + j, N_DEV)
            r_rdma = pltpu.make_async_remote_copy(
                src_ref=amax_ref.at[other],
                dst_ref=amax_ref.at[other],
                send_sem=amax_send_sems.at[other],
                recv_sem=amax_recv_sems.at[other],
                device_id=other, device_id_type=pl.DeviceIdType.LOGICAL,
            )
            r_rdma.wait_recv()
        for s_rdma in sends:
            s_rdma.wait_send()
        run_amax = jnp.max(amax_ref[...])

        scale = run_amax / 448.0
        inv_scale = 448.0 / run_amax
        y = out_ref[...]
        q = jnp.minimum(y * inv_scale, 448.0).astype(jnp.float8_e4m3fn)
        out_ref[...] = q.astype(jnp.float32) * scale

    return pl.pallas_call(
        body,
        out_shape=jax.ShapeDtypeStruct((N_DEV * m_per, n_per), jnp.float32),
        in_specs=[
            pl.BlockSpec(memory_space=pltpu.SMEM),
            pl.BlockSpec(memory_space=pltpu.VMEM),
            pl.BlockSpec(memory_space=pltpu.VMEM),
        ],
        out_specs=pl.BlockSpec(memory_space=pltpu.VMEM),
        scratch_shapes=[
            pltpu.VMEM((N_SLOT, half, k), jnp.float32),
            pltpu.SemaphoreType.DMA((N_SLOT,)),
            pltpu.SemaphoreType.DMA((N_SLOT,)),
            pltpu.VMEM((N_SLOT, half, k), jnp.float32),
            pltpu.SemaphoreType.DMA((N_SLOT,)),
            pltpu.SemaphoreType.DMA((N_SLOT,)),
            pltpu.SemaphoreType.REGULAR((2,)),
            pltpu.VMEM((2, half, k), jnp.float32),
            pltpu.VMEM((2, half, k), jnp.float32),
            pltpu.SemaphoreType.DMA((2,)),
            pltpu.SemaphoreType.DMA((2,)),
            pltpu.VMEM((N_DEV, 8, 128), jnp.float32),
            pltpu.SemaphoreType.DMA((N_DEV,)),
            pltpu.SemaphoreType.DMA((N_DEV,)),
        ],
        compiler_params=pltpu.CompilerParams(
            collective_id=0, vmem_limit_bytes=100 * 1024 * 1024
        ),
    )(meta, x, w_mat)


# device time: 309328 ns/iter; 2.4936x vs baseline; 1.1456x over previous
import jax
import jax.numpy as jnp
import numpy as np
from jax import lax
from jax.experimental import pallas as pl
from jax.experimental.pallas import tpu as pltpu

N_DEV = 16
N_MSG = 12
N_SLOT = 5
N_RELAY = 6

M_POS = [7, 6, 13, 12, 11, 10, 1, 0, 15, 14, 5, 4, 3, 2, 9, 8]


def _ring_order() -> list[int]:
    try:
        coords = sorted(
            tuple(d.coords)
            for d in jax.devices()
            if getattr(d, "core_on_chip", 1) == 1
        )
    except Exception:
        return list(range(N_DEV))
    if len(coords) != N_DEV:
        return list(range(N_DEV))
    snake = []
    zs = sorted({c[2] for c in coords})
    for z in zs:
        plane = [c for c in coords if c[2] == z]
        ys = sorted({c[1] for c in plane})
        for yi, y in enumerate(ys):
            row = sorted((c for c in plane if c[1] == y), reverse=bool(yi % 2))
            snake.extend(row)
    logical = {c: i for i, c in enumerate(snake)}
    xs = {c[0] for c in coords}
    ys = {c[1] for c in coords}
    if not (xs == {0, 1} and ys == {0, 1} and len(zs) * 4 == N_DEV):
        return list(range(N_DEV))
    cycle = (
        [(0, 0, z) for z in zs]
        + [(0, 1, z) for z in reversed(zs)]
        + [(1, 1, z) for z in zs]
        + [(1, 0, z) for z in reversed(zs)]
    )
    return [logical[c] for c in cycle]


def kernel(x, w_mat):
    m_per, k = x.shape
    k2, n_per = w_mat.shape
    assert k == k2
    half = m_per // 2

    ring = np.asarray(_ring_order(), dtype=np.int32)
    inv = np.empty(N_DEV, dtype=np.int32)
    inv[ring] = np.arange(N_DEV, dtype=np.int32)

    my = lax.axis_index("i")
    r_pos = jnp.asarray(inv)[my]
    ring_j = jnp.asarray(ring)
    succ = ring_j[(r_pos + 1) % N_DEV]
    pred = ring_j[(r_pos - 1) % N_DEV]
    msg = jnp.arange(N_MSG, dtype=jnp.int32)
    origins_f = ring_j[jnp.mod(r_pos - 1 - msg // 2, N_DEV)]
    origins_b = ring_j[jnp.mod(r_pos + 1 + msg // 2, N_DEV)]
    partner = ring_j[jnp.asarray(np.asarray(M_POS, dtype=np.int32))[r_pos]]
    relay_cls = jnp.mod(r_pos, 4)
    relay_offs = jnp.where(
        relay_cls <= 1,
        jnp.asarray([7, 8, 9], dtype=jnp.int32),
        jnp.asarray([9, 8, 7], dtype=jnp.int32),
    )
    relay_origins = ring_j[jnp.mod(r_pos + relay_offs, N_DEV)]
    meta = jnp.concatenate(
        [succ[None], pred[None], origins_f, origins_b,
         partner[None], relay_cls[None], relay_origins]
    ).astype(jnp.int32)

    off_f = [(i % 2) * half for i in range(N_MSG)]
    off_b = [(1 - i % 2) * half for i in range(N_MSG)]

    def body(
        meta_ref,
        x_ref,
        w_ref,
        out_ref,
        staged,
        comm_f,
        f_send_sems,
        f_recv_sems,
        comm_b,
        b_send_sems,
        b_recv_sems,
        credit_sems,
        relay_in,
        relay_send_sems,
        relay_recv_sems,
        stage_sem,
        amax_ref,
        amax_send_sems,
        amax_recv_sems,
    ):
        succ_id = meta_ref[0]
        pred_id = meta_ref[1]
        partner_id = meta_ref[2 + 2 * N_MSG]
        cls = meta_ref[2 + 2 * N_MSG + 1]
        my_id = lax.axis_index("i")

        barrier_sem = pltpu.get_barrier_semaphore()
        for nbr in (succ_id, pred_id, partner_id):
            pl.semaphore_signal(
                barrier_sem, inc=1, device_id=nbr,
                device_id_type=pl.DeviceIdType.LOGICAL,
            )
        pl.semaphore_wait(barrier_sem, 3)

        dirs = {
            "f": dict(comm=comm_f, ssem=f_send_sems, rsem=f_recv_sems,
                      out_id=succ_id, in_id=pred_id, credit=0,
                      off=off_f, meta0=2),
            "b": dict(comm=comm_b, ssem=b_send_sems, rsem=b_recv_sems,
                      out_id=pred_id, in_id=succ_id, credit=1,
                      off=off_b, meta0=2 + N_MSG),
        }
        send_desc = {"f": {}, "b": {}}

        def start_send(d, j):
            dd = dirs[d]
            if j < 2:
                src = x_ref.at[pl.ds(dd["off"][0] if j == 0 else dd["off"][1],
                                     half)]
            else:
                src = dd["comm"].at[(j - 2) % N_SLOT]
            rdma = pltpu.make_async_remote_copy(
                src_ref=src,
                dst_ref=dd["comm"].at[j % N_SLOT],
                send_sem=dd["ssem"].at[j % N_SLOT],
                recv_sem=dd["rsem"].at[j % N_SLOT],
                device_id=dd["out_id"],
                device_id_type=pl.DeviceIdType.LOGICAL,
            )
            rdma.start()
            send_desc[d][j] = rdma

        def recv_desc(d, i):
            dd = dirs[d]
            return pltpu.make_async_remote_copy(
                src_ref=dd["comm"].at[i % N_SLOT],
                dst_ref=dd["comm"].at[i % N_SLOT],
                send_sem=dd["ssem"].at[i % N_SLOT],
                recv_sem=dd["rsem"].at[i % N_SLOT],
                device_id=dd["in_id"],
                device_id_type=pl.DeviceIdType.LOGICAL,
            )

        def step_comm(d, i):
            dd = dirs[d]
            recv_desc(d, i).wait_recv()
            send_desc[d][i].wait_send()
            if 2 <= i <= 2 + (N_MSG - N_SLOT - 1):
                pl.semaphore_signal(
                    credit_sems.at[dd["credit"]], inc=1,
                    device_id=dd["in_id"],
                    device_id_type=pl.DeviceIdType.LOGICAL,
                )
            if i + 2 < N_MSG:
                if i + 2 >= N_SLOT:
                    pl.semaphore_wait(credit_sems.at[dd["credit"]], 1)
                start_send(d, i + 2)

        def gemm_store(chunk, row_start, amax):
            y = jnp.dot(chunk, w_ref[...], preferred_element_type=jnp.float32)
            y = jnp.maximum(y, 0.0)
            out_ref[pl.ds(row_start, chunk.shape[0]), :] = y
            return jnp.maximum(amax, jnp.max(y))

        for d in ("f", "b"):
            start_send(d, 0)
            start_send(d, 1)
        y0 = jnp.dot(x_ref[...], w_ref[...], preferred_element_type=jnp.float32)
        y0 = jnp.maximum(y0, 0.0)
        out_ref[pl.ds(my_id * m_per, m_per), :] = y0
        run_amax = jnp.max(y0)

        relay_desc = [
            pltpu.make_async_remote_copy(
                src_ref=staged.at[t],
                dst_ref=relay_in.at[t],
                send_sem=relay_send_sems.at[t],
                recv_sem=relay_recv_sems.at[t],
                device_id=partner_id,
                device_id_type=pl.DeviceIdType.LOGICAL,
            )
            for t in range(N_RELAY)
        ]
        x_a = x_ref.at[pl.ds(0, half)]
        x_b = x_ref.at[pl.ds(half, half)]

        def fs(m):
            return comm_f.at[m % N_SLOT]

        def bs(m):
            return comm_b.at[m % N_SLOT]

        stage_plan = {
            0: [(x_a, 1), (x_b, 1), (fs(0), 1), (fs(1), 1),
                (fs(2), 3), (fs(3), 3)],
            1: [(fs(2), 3), (fs(3), 3), (fs(4), 5),
                (fs(5), 5), (fs(6), 7), (fs(7), 7)],
            2: [(bs(3), 3), (bs(2), 3), (bs(5), 5),
                (bs(4), 5), (bs(7), 7), (bs(6), 7)],
            3: [(x_a, 1), (x_b, 1), (bs(1), 1), (bs(0), 1),
                (bs(3), 3), (bs(2), 3)],
        }

        for i in range(N_MSG):
            step_comm("f", i)
            step_comm("b", i)
            if i in (1, 3, 5, 7):
                for c in range(4):
                    due = [
                        (t, src)
                        for t, (src, rdy) in enumerate(stage_plan[c])
                        if rdy == i
                    ]
                    if due:
                        @pl.when(cls == c)
                        def _(due=due):
                            for t, src in due:
                                cp = pltpu.make_async_copy(
                                    src, staged.at[t], stage_sem
                                )
                                cp.start()
                                cp.wait()
            if i in (3, 5, 7):
                for t in (i - 3, i - 2):
                    relay_desc[t].start()
            if i >= N_MSG - N_RELAY:
                t = i - (N_MSG - N_RELAY)
                relay_desc[t].wait_recv()
                ro = meta_ref[2 + 2 * N_MSG + 2 + t // 2]
                run_amax = gemm_store(
                    relay_in[t], ro * m_per + (t % 2) * half, run_amax
                )
            of = meta_ref[2 + i]
            run_amax = gemm_store(
                comm_f[i % N_SLOT], of * m_per + off_f[i], run_amax
            )
            ob = meta_ref[2 + N_MSG + i]
            run_amax = gemm_store(
                comm_b[i % N_SLOT], ob * m_per + off_b[i], run_amax
            )

        for t in range(N_RELAY):
            relay_desc[t].wait_send()

        run_amax = jnp.maximum(run_amax, 0.0)

        amax_ref[my_id] = jnp.full((8, 128), run_amax, jnp.float32)
        sends = []
        for j in range(1, N_DEV):
            other = jnp.mod(my_id + j, N_DEV)
            s_rdma = pltpu.make_async_remote_copy(
                src_ref=amax_ref.at[my_id],
                dst_ref=amax_ref.at[my_id],
                send_sem=amax_send_sems.at[other],
                recv_sem=amax_recv_sems.at[my_id],
                device_id=other, device_id_type=pl.DeviceIdType.LOGICAL,
            )
            s_rdma.start()
            sends.append(s_rdma)
        for j in range(1, N_DEV):
            other = jnp.mod(my_id + j, N_DEV)
            r_rdma = pltpu.make_async_remote_copy(
                src_ref=amax_ref.at[other],
                dst_ref=amax_ref.at[other],
                send_sem=amax_send_sems.at[other],
                recv_sem=amax_recv_sems.at[other],
                device_id=other, device_id_type=pl.DeviceIdType.LOGICAL,
            )
            r_rdma.wait_recv()
        for s_rdma in sends:
            s_rdma.wait_send()
        run_amax = jnp.max(amax_ref[...])

        scale = run_amax / 448.0
        inv_scale = 448.0 / run_amax
        y = out_ref[...]
        q = jnp.minimum(y * inv_scale, 448.0).astype(jnp.float8_e4m3fn)
        out_ref[...] = q.astype(jnp.float32) * scale

    out, _ = pl.pallas_call(
        body,
        out_shape=(
            jax.ShapeDtypeStruct((N_DEV * m_per, n_per), jnp.float32),
            jax.ShapeDtypeStruct((N_RELAY, half, k), jnp.float32),
        ),
        in_specs=[
            pl.BlockSpec(memory_space=pltpu.SMEM),
            pl.BlockSpec(memory_space=pltpu.VMEM),
            pl.BlockSpec(memory_space=pltpu.VMEM),
        ],
        out_specs=(
            pl.BlockSpec(memory_space=pltpu.VMEM),
            pl.BlockSpec(memory_space=pltpu.MemorySpace.HBM),
        ),
        scratch_shapes=[
            pltpu.VMEM((N_SLOT, half, k), jnp.float32),
            pltpu.SemaphoreType.DMA((N_SLOT,)),
            pltpu.SemaphoreType.DMA((N_SLOT,)),
            pltpu.VMEM((N_SLOT, half, k), jnp.float32),
            pltpu.SemaphoreType.DMA((N_SLOT,)),
            pltpu.SemaphoreType.DMA((N_SLOT,)),
            pltpu.SemaphoreType.REGULAR((2,)),
            pltpu.VMEM((N_RELAY, half, k), jnp.float32),
            pltpu.SemaphoreType.DMA((N_RELAY,)),
            pltpu.SemaphoreType.DMA((N_RELAY,)),
            pltpu.SemaphoreType.DMA,
            pltpu.VMEM((N_DEV, 8, 128), jnp.float32),
            pltpu.SemaphoreType.DMA((N_DEV,)),
            pltpu.SemaphoreType.DMA((N_DEV,)),
        ],
        compiler_params=pltpu.CompilerParams(
            collective_id=0, vmem_limit_bytes=100 * 1024 * 1024
        ),
    )(meta, x, w_mat)
    return out


# device time: 278282 ns/iter; 2.7718x vs baseline; 1.1116x over previous
import jax
import jax.numpy as jnp
import numpy as np
from jax import lax
from jax.experimental import pallas as pl
from jax.experimental.pallas import tpu as pltpu

N_DEV = 16
N_MSG = 10
N_SLOT = 5
N_RELAY = 10

M_POS = [7, 6, 13, 12, 11, 10, 1, 0, 15, 14, 5, 4, 3, 2, 9, 8]


def _ring_order() -> list[int]:
    try:
        coords = sorted(
            tuple(d.coords)
            for d in jax.devices()
            if getattr(d, "core_on_chip", 1) == 1
        )
    except Exception:
        return list(range(N_DEV))
    if len(coords) != N_DEV:
        return list(range(N_DEV))
    snake = []
    zs = sorted({c[2] for c in coords})
    for z in zs:
        plane = [c for c in coords if c[2] == z]
        ys = sorted({c[1] for c in plane})
        for yi, y in enumerate(ys):
            row = sorted((c for c in plane if c[1] == y), reverse=bool(yi % 2))
            snake.extend(row)
    logical = {c: i for i, c in enumerate(snake)}
    xs = {c[0] for c in coords}
    ys = {c[1] for c in coords}
    if not (xs == {0, 1} and ys == {0, 1} and len(zs) * 4 == N_DEV):
        return list(range(N_DEV))
    cycle = (
        [(0, 0, z) for z in zs]
        + [(0, 1, z) for z in reversed(zs)]
        + [(1, 1, z) for z in zs]
        + [(1, 0, z) for z in reversed(zs)]
    )
    return [logical[c] for c in cycle]


def kernel(x, w_mat):
    m_per, k = x.shape
    k2, n_per = w_mat.shape
    assert k == k2
    half = m_per // 2

    ring = np.asarray(_ring_order(), dtype=np.int32)
    inv = np.empty(N_DEV, dtype=np.int32)
    inv[ring] = np.arange(N_DEV, dtype=np.int32)

    my = lax.axis_index("i")
    r_pos = jnp.asarray(inv)[my]
    ring_j = jnp.asarray(ring)
    succ = ring_j[(r_pos + 1) % N_DEV]
    pred = ring_j[(r_pos - 1) % N_DEV]
    msg = jnp.arange(N_MSG, dtype=jnp.int32)
    origins_f = ring_j[jnp.mod(r_pos - 1 - msg // 2, N_DEV)]
    origins_b = ring_j[jnp.mod(r_pos + 1 + msg // 2, N_DEV)]
    relay_cls = jnp.mod(r_pos, 4)
    a_tab, b_tab = [], []
    for p in range(N_DEV):
        c = p % 4
        if c in (0, 3):
            a_tab.append(M_POS[p])
            b_tab.append(M_POS[p])
        elif c == 1:
            a_tab.append((p + 5) % N_DEV)
            b_tab.append((p - 3) % N_DEV)
        else:
            a_tab.append((p - 5) % N_DEV)
            b_tab.append((p + 3) % N_DEV)
    peer_a = ring_j[jnp.asarray(np.asarray(a_tab, dtype=np.int32))[r_pos]]
    peer_b = ring_j[jnp.asarray(np.asarray(b_tab, dtype=np.int32))[r_pos]]
    offs_tab = jnp.asarray(
        [[7, 6, 8, 9, 10],
         [6, 7, 8, 10, 9],
         [10, 9, 8, 6, 7],
         [9, 10, 8, 7, 6]], dtype=jnp.int32,
    )
    relay_origins = ring_j[jnp.mod(r_pos + offs_tab[relay_cls], N_DEV)]
    meta = jnp.concatenate(
        [succ[None], pred[None], origins_f, origins_b,
         peer_a[None], peer_b[None], relay_cls[None], relay_origins]
    ).astype(jnp.int32)

    off_f = [(i % 2) * half for i in range(N_MSG)]
    off_b = [(1 - i % 2) * half for i in range(N_MSG)]

    def body(
        meta_ref,
        x_ref,
        w_ref,
        out_ref,
        staged,
        relay_in,
        comm_f,
        f_send_sems,
        f_recv_sems,
        comm_b,
        b_send_sems,
        b_recv_sems,
        credit_sems,
        relay_send_sems,
        relay_recv_sems,
        stage_sem,
        bounce,
        amax_ref,
        amax_send_sems,
        amax_recv_sems,
    ):
        succ_id = meta_ref[0]
        pred_id = meta_ref[1]
        a_id = meta_ref[2 + 2 * N_MSG]
        b_id = meta_ref[2 + 2 * N_MSG + 1]
        cls = meta_ref[2 + 2 * N_MSG + 2]
        my_id = lax.axis_index("i")

        barrier_sem = pltpu.get_barrier_semaphore()
        for nbr in (succ_id, pred_id, a_id, b_id):
            pl.semaphore_signal(
                barrier_sem, inc=1, device_id=nbr,
                device_id_type=pl.DeviceIdType.LOGICAL,
            )
        pl.semaphore_wait(barrier_sem, 4)

        dirs = {
            "f": dict(comm=comm_f, ssem=f_send_sems, rsem=f_recv_sems,
                      out_id=succ_id, in_id=pred_id, credit=0,
                      off=off_f, meta0=2),
            "b": dict(comm=comm_b, ssem=b_send_sems, rsem=b_recv_sems,
                      out_id=pred_id, in_id=succ_id, credit=1,
                      off=off_b, meta0=2 + N_MSG),
        }
        send_desc = {"f": {}, "b": {}}

        def start_send(d, j):
            dd = dirs[d]
            if j < 2:
                src = x_ref.at[pl.ds(dd["off"][0] if j == 0 else dd["off"][1],
                                     half)]
            else:
                src = dd["comm"].at[(j - 2) % N_SLOT]
            rdma = pltpu.make_async_remote_copy(
                src_ref=src,
                dst_ref=dd["comm"].at[j % N_SLOT],
                send_sem=dd["ssem"].at[j % N_SLOT],
                recv_sem=dd["rsem"].at[j % N_SLOT],
                device_id=dd["out_id"],
                device_id_type=pl.DeviceIdType.LOGICAL,
            )
            rdma.start()
            send_desc[d][j] = rdma

        def recv_desc(d, i):
            dd = dirs[d]
            return pltpu.make_async_remote_copy(
                src_ref=dd["comm"].at[i % N_SLOT],
                dst_ref=dd["comm"].at[i % N_SLOT],
                send_sem=dd["ssem"].at[i % N_SLOT],
                recv_sem=dd["rsem"].at[i % N_SLOT],
                device_id=dd["in_id"],
                device_id_type=pl.DeviceIdType.LOGICAL,
            )

        def step_comm(d, i):
            dd = dirs[d]
            recv_desc(d, i).wait_recv()
            send_desc[d][i].wait_send()
            if 2 <= i <= 2 + (N_MSG - N_SLOT - 1):
                pl.semaphore_signal(
                    credit_sems.at[dd["credit"]], inc=1,
                    device_id=dd["in_id"],
                    device_id_type=pl.DeviceIdType.LOGICAL,
                )
            if i + 2 < N_MSG:
                if i + 2 >= N_SLOT:
                    pl.semaphore_wait(credit_sems.at[dd["credit"]], 1)
                start_send(d, i + 2)

        def gemm_store(chunk, row_start, amax):
            y = jnp.dot(chunk, w_ref[...], preferred_element_type=jnp.float32)
            y = jnp.maximum(y, 0.0)
            out_ref[pl.ds(row_start, chunk.shape[0]), :] = y
            return jnp.maximum(amax, jnp.max(y))

        def consume_relay(t, amax):
            relay_desc[t].wait_recv()
            cp = pltpu.make_async_copy(
                relay_in.at[t], bounce.at[t % 2], stage_sem
            )
            cp.start()
            cp.wait()
            ro = meta_ref[2 + 2 * N_MSG + 3 + t // 2]
            return gemm_store(
                bounce[t % 2], ro * m_per + (t % 2) * half, amax
            )

        relay_desc = [
            pltpu.make_async_remote_copy(
                src_ref=staged.at[t],
                dst_ref=relay_in.at[t],
                send_sem=relay_send_sems.at[t],
                recv_sem=relay_recv_sems.at[t],
                device_id=a_id if t < 6 else b_id,
                device_id_type=pl.DeviceIdType.LOGICAL,
            )
            for t in range(N_RELAY)
        ]
        x_a = x_ref.at[pl.ds(0, half)]
        x_b = x_ref.at[pl.ds(half, half)]

        def fs(m):
            return comm_f.at[m % N_SLOT]

        def bs(m):
            return comm_b.at[m % N_SLOT]

        stage_plan = {
            0: [(x_a, 0), (x_b, 0), (bs(1), 1), (bs(0), 1), (fs(0), 1),
                (fs(1), 1), (fs(2), 3), (fs(3), 3), (fs(4), 5), (fs(5), 5)],
            1: [(fs(0), 1), (fs(1), 1), (fs(2), 3), (fs(3), 3), (fs(4), 5),
                (fs(5), 5), (bs(5), 5), (bs(4), 5), (bs(7), 7), (bs(6), 7)],
            2: [(bs(1), 1), (bs(0), 1), (bs(3), 3), (bs(2), 3), (bs(5), 5),
                (bs(4), 5), (fs(4), 5), (fs(5), 5), (fs(6), 7), (fs(7), 7)],
            3: [(x_a, 0), (x_b, 0), (fs(0), 1), (fs(1), 1), (bs(1), 1),
                (bs(0), 1), (bs(3), 3), (bs(2), 3), (bs(5), 5), (bs(4), 5)],
        }

        def stage_and_send(site):
            for c in range(4):
                due = [
                    (t, src)
                    for t, (src, rdy) in enumerate(stage_plan[c])
                    if rdy == site
                ]
                if due:
                    @pl.when(cls == c)
                    def _(due=due):
                        for t, src in due:
                            cp = pltpu.make_async_copy(
                                src, staged.at[t], stage_sem
                            )
                            cp.start()
                            cp.wait()
                            relay_desc[t].start()

        for d in ("f", "b"):
            start_send(d, 0)
            start_send(d, 1)
        stage_and_send(0)
        y0 = jnp.dot(x_ref[...], w_ref[...], preferred_element_type=jnp.float32)
        y0 = jnp.maximum(y0, 0.0)
        out_ref[pl.ds(my_id * m_per, m_per), :] = y0
        run_amax = jnp.max(y0)

        for i in range(N_MSG):
            step_comm("f", i)
            step_comm("b", i)
            if i in (1, 3, 5, 7):
                stage_and_send(i)
            if i >= 4:
                run_amax = consume_relay(i - 4, run_amax)
            of = meta_ref[2 + i]
            run_amax = gemm_store(
                comm_f[i % N_SLOT], of * m_per + off_f[i], run_amax
            )
            ob = meta_ref[2 + N_MSG + i]
            run_amax = gemm_store(
                comm_b[i % N_SLOT], ob * m_per + off_b[i], run_amax
            )

        for t in range(N_MSG - 4, N_RELAY):
            run_amax = consume_relay(t, run_amax)
        for t in range(N_RELAY):
            relay_desc[t].wait_send()

        run_amax = jnp.maximum(run_amax, 0.0)

        amax_ref[my_id] = jnp.full((8, 128), run_amax, jnp.float32)
        sends = []
        for j in range(1, N_DEV):
            other = jnp.mod(my_id + j, N_DEV)
            s_rdma = pltpu.make_async_remote_copy(
                src_ref=amax_ref.at[my_id],
                dst_ref=amax_ref.at[my_id],
                send_sem=amax_send_sems.at[other],
                recv_sem=amax_recv_sems.at[my_id],
                device_id=other, device_id_type=pl.DeviceIdType.LOGICAL,
            )
            s_rdma.start()
            sends.append(s_rdma)
        for j in range(1, N_DEV):
            other = jnp.mod(my_id + j, N_DEV)
            r_rdma = pltpu.make_async_remote_copy(
                src_ref=amax_ref.at[other],
                dst_ref=amax_ref.at[other],
                send_sem=amax_send_sems.at[other],
                recv_sem=amax_recv_sems.at[other],
                device_id=other, device_id_type=pl.DeviceIdType.LOGICAL,
            )
            r_rdma.wait_recv()
        for s_rdma in sends:
            s_rdma.wait_send()
        run_amax = jnp.max(amax_ref[...])

        scale = run_amax / 448.0
        inv_scale = 448.0 / run_amax
        y = out_ref[...]
        q = jnp.minimum(y * inv_scale, 448.0).astype(jnp.float8_e4m3fn)
        out_ref[...] = q.astype(jnp.float32) * scale

    out = pl.pallas_call(
        body,
        out_shape=(
            jax.ShapeDtypeStruct((N_DEV * m_per, n_per), jnp.float32),
            jax.ShapeDtypeStruct((N_RELAY, half, k), jnp.float32),
            jax.ShapeDtypeStruct((N_RELAY, half, k), jnp.float32),
        ),
        in_specs=[
            pl.BlockSpec(memory_space=pltpu.SMEM),
            pl.BlockSpec(memory_space=pltpu.VMEM),
            pl.BlockSpec(memory_space=pltpu.VMEM),
        ],
        out_specs=(
            pl.BlockSpec(memory_space=pltpu.VMEM),
            pl.BlockSpec(memory_space=pltpu.MemorySpace.HBM),
            pl.BlockSpec(memory_space=pltpu.MemorySpace.HBM),
        ),
        scratch_shapes=[
            pltpu.VMEM((N_SLOT, half, k), jnp.float32),
            pltpu.SemaphoreType.DMA((N_SLOT,)),
            pltpu.SemaphoreType.DMA((N_SLOT,)),
            pltpu.VMEM((N_SLOT, half, k), jnp.float32),
            pltpu.SemaphoreType.DMA((N_SLOT,)),
            pltpu.SemaphoreType.DMA((N_SLOT,)),
            pltpu.SemaphoreType.REGULAR((2,)),
            pltpu.SemaphoreType.DMA((N_RELAY,)),
            pltpu.SemaphoreType.DMA((N_RELAY,)),
            pltpu.SemaphoreType.DMA,
            pltpu.VMEM((2, half, k), jnp.float32),
            pltpu.VMEM((N_DEV, 8, 128), jnp.float32),
            pltpu.SemaphoreType.DMA((N_DEV,)),
            pltpu.SemaphoreType.DMA((N_DEV,)),
        ],
        compiler_params=pltpu.CompilerParams(
            collective_id=0, vmem_limit_bytes=100 * 1024 * 1024
        ),
    )(meta, x, w_mat)
    return out[0]
